# baseline (speedup 1.0000x reference)
import functools

import jax
import jax.numpy as jnp
from jax import lax
from jax.experimental import pallas as pl
from jax.experimental.pallas import tpu as pltpu
from jax.experimental.pallas import tpu_sc as plsc

NUM_CLASS = 100000
CTX_DIM = 512
N_CLS_CTX = 4
N_PRE = 5
N_SUF = 68
N_BCAST = N_PRE + N_SUF
CLIP_LEN = 77
BATCH = 4096
LANES = 16
VPR = CTX_DIM // LANES

NC = 2
NS = 16
NW = NC * NS
BPW = BATCH // NW
K = 8
SLICE = 128
NSLICE = BATCH // SLICE
UPW = (N_BCAST * NSLICE) // NW


@functools.partial(
    pl.kernel,
    mesh=plsc.VectorSubcoreMesh(core_axis_name="c", subcore_axis_name="s"),
    out_type=jax.ShapeDtypeStruct((CLIP_LEN, BATCH, CTX_DIM), jnp.float32),
    scratch_types=[
        pltpu.VMEM((BPW,), jnp.int32),
        pltpu.VMEM((K, N_CLS_CTX, CTX_DIM), jnp.float32),
        pltpu.VMEM((N_CLS_CTX, K, CTX_DIM), jnp.float32),
        pltpu.VMEM((1, SLICE, CTX_DIM), jnp.float32),
        pltpu.VMEM((8, CTX_DIM), jnp.float32),
        pltpu.SemaphoreType.DMA,
        pltpu.SemaphoreType.DMA,
        pltpu.SemaphoreType.DMA,
        pltpu.SemaphoreType.DMA,
    ],
)
def _prompt_assemble(label_h, cls_h, ps_h, out_h,
                     idx_v, cls_v, clsT_v, rep_v, stage_v, gsem, csem, bsem, hsem):
    cid = lax.axis_index("c")
    sid = lax.axis_index("s")
    wid = sid * NC + cid
    base = wid * BPW

    u0 = wid * UPW
    tb_lo = u0 // NSLICE
    tb_hi = (u0 + UPW - 1) // NSLICE

    def k_bounds(tb):
        klo = jnp.maximum(u0 - tb * NSLICE, 0)
        khi = jnp.minimum(u0 + UPW - tb * NSLICE, NSLICE)
        return klo, khi

    def fill_rep(tb):
        pltpu.sync_copy(ps_h.at[pl.ds((tb // 8) * 8, 8)], stage_v)
        tbm = tb % 8
        rows = [stage_v[tbm, pl.ds(v * LANES, LANES)] for v in range(VPR)]

        def fill(rw, _):
            for v in range(VPR):
                rep_v[0, rw, pl.ds(v * LANES, LANES)] = rows[v]
            return None

        lax.fori_loop(0, SLICE, fill, None)

    def issue_tb(tb):
        t = jnp.where(tb < N_PRE, tb, tb + N_CLS_CTX)
        klo, khi = k_bounds(tb)
        pltpu.async_copy(
            rep_v, out_h.at[pl.ds(t, 1), pl.ds(klo * SLICE, SLICE)], bsem).wait()
        seed = out_h.at[pl.ds(t, 1), pl.ds(klo * SLICE, SLICE)]

        def issue(k, _):
            dst = out_h.at[pl.ds(t, 1), pl.ds(k * SLICE, SLICE)]

            @pl.when((k - klo) % 2 == 1)
            def _copy():
                pltpu.async_copy(seed, dst, hsem)

            @pl.when((k - klo) % 2 == 0)
            def _stream():
                pltpu.async_copy(rep_v, dst, bsem)

            return None

        lax.fori_loop(klo + 1, khi, issue, None)

    def drain_tb(tb):
        klo, khi = k_bounds(tb)
        n = khi - klo - 1
        n_copy = (n + 1) // 2
        n_stream = n - n_copy

        def drain_s(k, _):
            pltpu.make_async_copy(
                rep_v, out_h.at[pl.ds(0, 1), pl.ds(0, SLICE)], bsem).wait()
            return None

        lax.fori_loop(0, n_stream, drain_s, None)

        def drain_h(k, _):
            pltpu.make_async_copy(
                out_h.at[pl.ds(0, 1), pl.ds(0, SLICE)],
                out_h.at[pl.ds(0, 1), pl.ds(SLICE, SLICE)], hsem).wait()
            return None

        lax.fori_loop(0, n_copy, drain_h, None)

    pltpu.sync_copy(label_h.at[pl.ds(base, BPW)], idx_v)

    fill_rep(tb_lo)
    issue_tb(tb_lo)

    def chunk(c, _):
        pltpu.async_copy(cls_h.at[idx_v.at[pl.ds(c * K, K)]], cls_v, gsem).wait()

        @pl.when(c > 0)
        def _drain_prev():
            for r in range(N_CLS_CTX):
                pltpu.make_async_copy(
                    clsT_v.at[pl.ds(r, 1)],
                    out_h.at[pl.ds(N_PRE + r, 1), pl.ds(base, K)], csem).wait()

        def tr(j, _):
            for r in range(N_CLS_CTX):
                for v in range(VPR):
                    clsT_v[r, j, pl.ds(v * LANES, LANES)] = (
                        cls_v[j, r, pl.ds(v * LANES, LANES)])
            return None

        lax.fori_loop(0, K, tr, None)
        for r in range(N_CLS_CTX):
            pltpu.async_copy(
                clsT_v.at[pl.ds(r, 1)],
                out_h.at[pl.ds(N_PRE + r, 1), pl.ds(base + c * K, K)], csem)
        return None

    lax.fori_loop(0, BPW // K, chunk, None)

    def per_tb(tb, _):
        drain_tb(tb - 1)
        fill_rep(tb)
        issue_tb(tb)
        return None

    lax.fori_loop(tb_lo + 1, tb_hi + 1, per_tb, None)

    drain_tb(tb_hi)
    for r in range(N_CLS_CTX):
        pltpu.make_async_copy(
            clsT_v.at[pl.ds(r, 1)],
            out_h.at[pl.ds(N_PRE + r, 1), pl.ds(base, K)], csem).wait()


def kernel(label, cls_ctx, token_prefix, token_suffix):
    ps = jnp.concatenate([token_prefix, token_suffix], axis=1)
    ps = jnp.pad(ps, ((0, 0), (0, 80 - N_BCAST), (0, 0))).reshape(80, CTX_DIM)
    out = _prompt_assemble(label.astype(jnp.int32), cls_ctx, ps)
    return out.transpose(1, 0, 2)


# speedup vs baseline: 33.1610x; 33.1610x over previous
import functools

import jax
import jax.numpy as jnp
from jax import lax
from jax.experimental import pallas as pl
from jax.experimental.pallas import tpu as pltpu
from jax.experimental.pallas import tpu_sc as plsc

NUM_CLASS = 100000
CTX_DIM = 512
N_CLS_CTX = 4
N_PRE = 5
N_SUF = 68
N_BCAST = N_PRE + N_SUF
CLIP_LEN = 77
BATCH = 4096
LANES = 16
VPR = CTX_DIM // LANES

NC = 2
NS = 16
NW = NC * NS
BPW = BATCH // NW
K = 8
SLICE = 32
NSLICE = BATCH // SLICE
UPW = (N_BCAST * NSLICE) // NW


@functools.partial(
    pl.kernel,
    mesh=plsc.VectorSubcoreMesh(core_axis_name="c", subcore_axis_name="s"),
    out_type=jax.ShapeDtypeStruct((CLIP_LEN, BATCH, CTX_DIM), jnp.float32),
    scratch_types=[
        pltpu.VMEM((BPW,), jnp.int32),
        pltpu.VMEM((K, N_CLS_CTX, CTX_DIM), jnp.float32),
        pltpu.VMEM((N_CLS_CTX, K, CTX_DIM), jnp.float32),
        pltpu.VMEM((1, SLICE, CTX_DIM), jnp.float32),
        pltpu.VMEM((8, CTX_DIM), jnp.float32),
        pltpu.VMEM_SHARED((NS, SLICE, CTX_DIM), jnp.float32),
        pltpu.SemaphoreType.DMA,
        pltpu.SemaphoreType.DMA,
        pltpu.SemaphoreType.DMA,
        pltpu.SemaphoreType.DMA,
    ],
)
def _prompt_assemble(label_h, cls_h, ps_h, out_h,
                     idx_v, cls_v, clsT_v, rep_v, stage_v, shr_v, gsem, csem, bsem, ssem):
    cid = lax.axis_index("c")
    sid = lax.axis_index("s")
    wid = sid * NC + cid
    base = wid * BPW

    u0 = wid * UPW
    tb_lo = u0 // NSLICE
    tb_hi = (u0 + UPW - 1) // NSLICE

    def k_bounds(tb):
        klo = jnp.maximum(u0 - tb * NSLICE, 0)
        khi = jnp.minimum(u0 + UPW - tb * NSLICE, NSLICE)
        return klo, khi

    def fill_rep(tb):
        pltpu.sync_copy(ps_h.at[pl.ds((tb // 8) * 8, 8)], stage_v)
        tbm = tb % 8
        rows = [stage_v[tbm, pl.ds(v * LANES, LANES)] for v in range(VPR)]

        def fill(rw, _):
            for v in range(VPR):
                rep_v[0, rw, pl.ds(v * LANES, LANES)] = rows[v]
            return None

        lax.fori_loop(0, SLICE, fill, None)

    def issue_tb(tb, slot):
        t = jnp.where(tb < N_PRE, tb, tb + N_CLS_CTX)
        klo, khi = k_bounds(tb)
        pltpu.sync_copy(rep_v, slot)

        def issue(k, _):
            dst = out_h.at[pl.ds(t, 1), pl.ds(k * SLICE, SLICE)]

            @pl.when((k - klo) % 2 == 1)
            def _from_spmem():
                pltpu.async_copy(slot, dst, ssem)

            @pl.when((k - klo) % 2 == 0)
            def _from_tile():
                pltpu.async_copy(rep_v, dst, bsem)

            return None

        lax.fori_loop(klo, khi, issue, None)

    def drain_tb(tb, slot):
        klo, khi = k_bounds(tb)
        n = khi - klo
        n_spmem = n // 2
        n_tile = n - n_spmem

        def drain_t(k, _):
            pltpu.make_async_copy(
                rep_v, out_h.at[pl.ds(0, 1), pl.ds(0, SLICE)], bsem).wait()
            return None

        lax.fori_loop(0, n_tile, drain_t, None)

        def drain_s(k, _):
            pltpu.make_async_copy(
                slot, out_h.at[pl.ds(0, 1), pl.ds(0, SLICE)], ssem).wait()
            return None

        lax.fori_loop(0, n_spmem, drain_s, None)

    pltpu.sync_copy(label_h.at[pl.ds(base, BPW)], idx_v)
    slot = shr_v.at[pl.ds(sid, 1)]

    fill_rep(tb_lo)
    issue_tb(tb_lo, slot)

    def chunk(c, _):
        pltpu.async_copy(cls_h.at[idx_v.at[pl.ds(c * K, K)]], cls_v, gsem).wait()

        @pl.when(c > 0)
        def _drain_prev():
            for r in range(N_CLS_CTX):
                pltpu.make_async_copy(
                    clsT_v.at[pl.ds(r, 1)],
                    out_h.at[pl.ds(N_PRE + r, 1), pl.ds(base, K)], csem).wait()

        def tr(j, _):
            for r in range(N_CLS_CTX):
                for v in range(VPR):
                    clsT_v[r, j, pl.ds(v * LANES, LANES)] = (
                        cls_v[j, r, pl.ds(v * LANES, LANES)])
            return None

        lax.fori_loop(0, K, tr, None)
        for r in range(N_CLS_CTX):
            pltpu.async_copy(
                clsT_v.at[pl.ds(r, 1)],
                out_h.at[pl.ds(N_PRE + r, 1), pl.ds(base + c * K, K)], csem)
        return None

    lax.fori_loop(0, BPW // K, chunk, None)

    def per_tb(tb, _):
        drain_tb(tb - 1, slot)
        fill_rep(tb)
        issue_tb(tb, slot)
        return None

    lax.fori_loop(tb_lo + 1, tb_hi + 1, per_tb, None)

    drain_tb(tb_hi, slot)
    for r in range(N_CLS_CTX):
        pltpu.make_async_copy(
            clsT_v.at[pl.ds(r, 1)],
            out_h.at[pl.ds(N_PRE + r, 1), pl.ds(base, K)], csem).wait()


def kernel(label, cls_ctx, token_prefix, token_suffix):
    ps = jnp.concatenate([token_prefix, token_suffix], axis=1)
    ps = jnp.pad(ps, ((0, 0), (0, 80 - N_BCAST), (0, 0))).reshape(80, CTX_DIM)
    out = _prompt_assemble(label.astype(jnp.int32), cls_ctx, ps)
    return out.transpose(1, 0, 2)


# speedup vs baseline: 33.3668x; 1.0062x over previous
import functools

import jax
import jax.numpy as jnp
from jax import lax
from jax.experimental import pallas as pl
from jax.experimental.pallas import tpu as pltpu
from jax.experimental.pallas import tpu_sc as plsc

NUM_CLASS = 100000
CTX_DIM = 512
N_CLS_CTX = 4
N_PRE = 5
N_SUF = 68
N_BCAST = N_PRE + N_SUF
CLIP_LEN = 77
BATCH = 4096
LANES = 16
VPR = CTX_DIM // LANES

NC = 2
NS = 16
NW = NC * NS
BPW = BATCH // NW
K = 8
SLICE = 64
NSLICE = BATCH // SLICE
UPW = (N_BCAST * NSLICE) // NW


@functools.partial(
    pl.kernel,
    mesh=plsc.VectorSubcoreMesh(core_axis_name="c", subcore_axis_name="s"),
    out_type=jax.ShapeDtypeStruct((CLIP_LEN, BATCH, CTX_DIM), jnp.float32),
    scratch_types=[
        pltpu.VMEM((BPW,), jnp.int32),
        pltpu.VMEM((K, N_CLS_CTX, CTX_DIM), jnp.float32),
        pltpu.VMEM((N_CLS_CTX, K, CTX_DIM), jnp.float32),
        pltpu.VMEM((1, SLICE, CTX_DIM), jnp.float32),
        pltpu.VMEM((8, CTX_DIM), jnp.float32),
        pltpu.VMEM_SHARED((NS, SLICE, CTX_DIM), jnp.float32),
        pltpu.SemaphoreType.DMA,
        pltpu.SemaphoreType.DMA,
        pltpu.SemaphoreType.DMA,
        pltpu.SemaphoreType.DMA,
    ],
)
def _prompt_assemble(label_h, cls_h, ps_h, out_h,
                     idx_v, cls_v, clsT_v, rep_v, stage_v, shr_v, gsem, csem, bsem, ssem):
    cid = lax.axis_index("c")
    sid = lax.axis_index("s")
    wid = sid * NC + cid
    base = wid * BPW

    u0 = wid * UPW
    tb_lo = u0 // NSLICE
    tb_hi = (u0 + UPW - 1) // NSLICE

    def k_bounds(tb):
        klo = jnp.maximum(u0 - tb * NSLICE, 0)
        khi = jnp.minimum(u0 + UPW - tb * NSLICE, NSLICE)
        return klo, khi

    def fill_rep(tb):
        pltpu.sync_copy(ps_h.at[pl.ds((tb // 8) * 8, 8)], stage_v)
        tbm = tb % 8
        rows = [stage_v[tbm, pl.ds(v * LANES, LANES)] for v in range(VPR)]

        def fill(rw, _):
            for v in range(VPR):
                rep_v[0, rw, pl.ds(v * LANES, LANES)] = rows[v]
            return None

        lax.fori_loop(0, SLICE, fill, None)

    def issue_tb(tb, slot):
        t = jnp.where(tb < N_PRE, tb, tb + N_CLS_CTX)
        klo, khi = k_bounds(tb)
        pltpu.sync_copy(rep_v, slot)

        def issue(k, _):
            dst = out_h.at[pl.ds(t, 1), pl.ds(k * SLICE, SLICE)]

            @pl.when((k - klo) % 2 == 1)
            def _from_spmem():
                pltpu.async_copy(slot, dst, ssem)

            @pl.when((k - klo) % 2 == 0)
            def _from_tile():
                pltpu.async_copy(rep_v, dst, bsem)

            return None

        lax.fori_loop(klo, khi, issue, None)

    def drain_tb(tb, slot):
        klo, khi = k_bounds(tb)
        n = khi - klo
        n_spmem = n // 2
        n_tile = n - n_spmem

        def drain_t(k, _):
            pltpu.make_async_copy(
                rep_v, out_h.at[pl.ds(0, 1), pl.ds(0, SLICE)], bsem).wait()
            return None

        lax.fori_loop(0, n_tile, drain_t, None)

        def drain_s(k, _):
            pltpu.make_async_copy(
                slot, out_h.at[pl.ds(0, 1), pl.ds(0, SLICE)], ssem).wait()
            return None

        lax.fori_loop(0, n_spmem, drain_s, None)

    pltpu.sync_copy(label_h.at[pl.ds(base, BPW)], idx_v)
    slot = shr_v.at[pl.ds(sid, 1)]

    fill_rep(tb_lo)
    issue_tb(tb_lo, slot)

    def chunk(c, _):
        pltpu.async_copy(cls_h.at[idx_v.at[pl.ds(c * K, K)]], cls_v, gsem).wait()

        @pl.when(c > 0)
        def _drain_prev():
            for r in range(N_CLS_CTX):
                pltpu.make_async_copy(
                    clsT_v.at[pl.ds(r, 1)],
                    out_h.at[pl.ds(N_PRE + r, 1), pl.ds(base, K)], csem).wait()

        def tr(j, _):
            for r in range(N_CLS_CTX):
                for v in range(VPR):
                    clsT_v[r, j, pl.ds(v * LANES, LANES)] = (
                        cls_v[j, r, pl.ds(v * LANES, LANES)])
            return None

        lax.fori_loop(0, K, tr, None)
        for r in range(N_CLS_CTX):
            pltpu.async_copy(
                clsT_v.at[pl.ds(r, 1)],
                out_h.at[pl.ds(N_PRE + r, 1), pl.ds(base + c * K, K)], csem)
        return None

    lax.fori_loop(0, BPW // K, chunk, None)

    def per_tb(tb, _):
        drain_tb(tb - 1, slot)
        fill_rep(tb)
        issue_tb(tb, slot)
        return None

    lax.fori_loop(tb_lo + 1, tb_hi + 1, per_tb, None)

    drain_tb(tb_hi, slot)
    for r in range(N_CLS_CTX):
        pltpu.make_async_copy(
            clsT_v.at[pl.ds(r, 1)],
            out_h.at[pl.ds(N_PRE + r, 1), pl.ds(base, K)], csem).wait()


def kernel(label, cls_ctx, token_prefix, token_suffix):
    ps = jnp.concatenate([token_prefix, token_suffix], axis=1)
    ps = jnp.pad(ps, ((0, 0), (0, 80 - N_BCAST), (0, 0))).reshape(80, CTX_DIM)
    out = _prompt_assemble(label.astype(jnp.int32), cls_ctx, ps)
    return out.transpose(1, 0, 2)


# speedup vs baseline: 35.0845x; 1.0515x over previous
import functools

import jax
import jax.numpy as jnp
from jax import lax
from jax.experimental import pallas as pl
from jax.experimental.pallas import tpu as pltpu
from jax.experimental.pallas import tpu_sc as plsc

NUM_CLASS = 100000
CTX_DIM = 512
N_CLS_CTX = 4
N_PRE = 5
N_SUF = 68
N_BCAST = N_PRE + N_SUF
CLIP_LEN = 77
BATCH = 4096
LANES = 16
VPR = CTX_DIM // LANES

NC = 2
NS = 16
NW = NC * NS
BPW = BATCH // NW
K = 8
NCHUNK = BPW // K
SLICE = 64
NSLICE = BATCH // SLICE
UPW = (N_BCAST * NSLICE) // NW


@functools.partial(
    pl.kernel,
    mesh=plsc.VectorSubcoreMesh(core_axis_name="c", subcore_axis_name="s"),
    out_type=jax.ShapeDtypeStruct((CLIP_LEN, BATCH, CTX_DIM), jnp.float32),
    scratch_types=[
        pltpu.VMEM((BPW,), jnp.int32),
        pltpu.VMEM((K, N_CLS_CTX, CTX_DIM), jnp.float32),
        pltpu.VMEM((N_CLS_CTX, K, CTX_DIM), jnp.float32),
        pltpu.VMEM((1, SLICE, CTX_DIM), jnp.float32),
        pltpu.VMEM((8, CTX_DIM), jnp.float32),
        pltpu.VMEM_SHARED((NS, SLICE, CTX_DIM), jnp.float32),
        pltpu.SemaphoreType.DMA,
        pltpu.SemaphoreType.DMA,
        pltpu.SemaphoreType.DMA,
        pltpu.SemaphoreType.DMA,
    ],
)
def _prompt_assemble(label_h, cls_h, ps_h, out_h,
                     idx_v, cls_v, clsT_v, rep_v, stage_v, shr_v,
                     gsem, csem, bsem, ssem):
    cid = lax.axis_index("c")
    sid = lax.axis_index("s")
    wid = sid * NC + cid
    base = wid * BPW

    u0 = wid * UPW
    tb_lo = u0 // NSLICE
    tb_hi = (u0 + UPW - 1) // NSLICE

    def k_bounds(tb):
        klo = jnp.maximum(u0 - tb * NSLICE, 0)
        khi = jnp.minimum(u0 + UPW - tb * NSLICE, NSLICE)
        return klo, khi

    def fill_rep(tb):
        pltpu.sync_copy(ps_h.at[pl.ds((tb // 8) * 8, 8)], stage_v)
        tbm = tb % 8
        rows = [stage_v[tbm, pl.ds(v * LANES, LANES)] for v in range(VPR)]

        def fill(rw, _):
            for v in range(VPR):
                rep_v[0, rw, pl.ds(v * LANES, LANES)] = rows[v]
            return None

        lax.fori_loop(0, SLICE, fill, None)

    def issue_tb(tb, slot):
        t = jnp.where(tb < N_PRE, tb, tb + N_CLS_CTX)
        klo, khi = k_bounds(tb)
        pltpu.sync_copy(rep_v, slot)

        def issue(k, _):
            dst = out_h.at[pl.ds(t, 1), pl.ds(k * SLICE, SLICE)]

            @pl.when((k - klo) % 2 == 1)
            def _from_spmem():
                pltpu.async_copy(slot, dst, ssem)

            @pl.when((k - klo) % 2 == 0)
            def _from_tile():
                pltpu.async_copy(rep_v, dst, bsem)

            return None

        lax.fori_loop(klo, khi, issue, None)

    def drain_tile(tb):
        klo, khi = k_bounds(tb)
        n = khi - klo

        def drain(k, _):
            pltpu.make_async_copy(
                rep_v, out_h.at[pl.ds(0, 1), pl.ds(0, SLICE)], bsem).wait()
            return None

        lax.fori_loop(0, (n + 1) // 2, drain, None)

    def drain_spmem(tb, slot):
        klo, khi = k_bounds(tb)
        n = khi - klo

        def drain(k, _):
            pltpu.make_async_copy(
                slot, out_h.at[pl.ds(0, 1), pl.ds(0, SLICE)], ssem).wait()
            return None

        lax.fori_loop(0, n // 2, drain, None)

    def cls_chunk(c, _):
        pltpu.async_copy(cls_h.at[idx_v.at[pl.ds(c * K, K)]], cls_v, gsem).wait()

        @pl.when(c > 0)
        def _drain_prev():
            for r in range(N_CLS_CTX):
                pltpu.make_async_copy(
                    clsT_v.at[pl.ds(r, 1)],
                    out_h.at[pl.ds(N_PRE + r, 1), pl.ds(base, K)], csem).wait()

        def tr(j, _):
            for r in range(N_CLS_CTX):
                for v in range(VPR):
                    clsT_v[r, j, pl.ds(v * LANES, LANES)] = (
                        cls_v[j, r, pl.ds(v * LANES, LANES)])
            return None

        lax.fori_loop(0, K, tr, None)
        for r in range(N_CLS_CTX):
            pltpu.async_copy(
                clsT_v.at[pl.ds(r, 1)],
                out_h.at[pl.ds(N_PRE + r, 1), pl.ds(base + c * K, K)], csem)
        return None

    pltpu.sync_copy(label_h.at[pl.ds(base, BPW)], idx_v)
    slot = shr_v.at[pl.ds(sid, 1)]

    fill_rep(tb_lo)
    issue_tb(tb_lo, slot)

    lax.fori_loop(0, NCHUNK // 2, cls_chunk, None)

    drain_tile(tb_lo)
    fill_rep(tb_lo + 1)
    drain_spmem(tb_lo, slot)
    issue_tb(tb_lo + 1, slot)

    lax.fori_loop(NCHUNK // 2, NCHUNK, cls_chunk, None)

    def per_tb(tb, _):
        drain_tile(tb - 1)
        fill_rep(tb)
        drain_spmem(tb - 1, slot)
        issue_tb(tb, slot)
        return None

    lax.fori_loop(tb_lo + 2, tb_hi + 1, per_tb, None)

    drain_tile(tb_hi)
    drain_spmem(tb_hi, slot)
    for r in range(N_CLS_CTX):
        pltpu.make_async_copy(
            clsT_v.at[pl.ds(r, 1)],
            out_h.at[pl.ds(N_PRE + r, 1), pl.ds(base, K)], csem).wait()


def kernel(label, cls_ctx, token_prefix, token_suffix):
    ps = jnp.concatenate([token_prefix, token_suffix], axis=1)
    ps = jnp.pad(ps, ((0, 0), (0, 80 - N_BCAST), (0, 0))).reshape(80, CTX_DIM)
    out = _prompt_assemble(label.astype(jnp.int32), cls_ctx, ps)
    return out.transpose(1, 0, 2)
